# baseline (speedup 1.0000x reference)
import functools

import jax
import jax.numpy as jnp
from jax import lax
from jax.experimental import pallas as pl
from jax.experimental.pallas import tpu as pltpu
from jax.experimental.pallas import tpu_sc as plsc

B = 16384
D = 32
L = 16

_info = plsc.get_sparse_core_info()
NC = _info.num_cores
NS = _info.num_subcores
NW = NC * NS
BPW = B // NW
CHUNK = 128
NCHUNK = BPW // CHUNK


def _sc_body(ids_hbm, eps_hbm, mu_hbm, lv_hbm,
             lat_out, mu_out, lv_out,
             idx_v, mu_v, lv_v, eps_v, lat_v,
             sem_mu, sem_lv, sem_eps, sem_out):
    wid = lax.axis_index("s") * NC + lax.axis_index("c")
    base = wid * BPW

    for j in range(NCHUNK):
        pltpu.sync_copy(ids_hbm.at[pl.ds(base + j * CHUNK, CHUNK)], idx_v.at[j])

    mu_cps = [
        pltpu.async_copy(mu_hbm.at[idx_v.at[j]],
                         mu_v.at[pl.ds(j * CHUNK, CHUNK)], sem_mu)
        for j in range(NCHUNK)
    ]
    lv_cps = [
        pltpu.async_copy(lv_hbm.at[idx_v.at[j]],
                         lv_v.at[pl.ds(j * CHUNK, CHUNK)], sem_lv)
        for j in range(NCHUNK)
    ]
    eps_cp = pltpu.async_copy(eps_hbm.at[pl.ds(base, BPW)], eps_v, sem_eps)

    eps_cp.wait()
    out_cps = []
    for j in range(NCHUNK):
        mu_cps[j].wait()
        lv_cps[j].wait()

        def body(i, carry):
            for h in range(D // L):
                sl = pl.ds(h * L, L)
                std = jnp.exp(lv_v[i, sl] * 0.5)
                lat_v[i, sl] = mu_v[i, sl] + eps_v[i, sl] * std
            return carry

        lax.fori_loop(j * CHUNK, (j + 1) * CHUNK, body, 0)

        row = pl.ds(j * CHUNK, CHUNK)
        orow = pl.ds(base + j * CHUNK, CHUNK)
        out_cps.append(pltpu.async_copy(mu_v.at[row], mu_out.at[orow], sem_out))
        out_cps.append(pltpu.async_copy(lv_v.at[row], lv_out.at[orow], sem_out))
        out_cps.append(pltpu.async_copy(lat_v.at[row], lat_out.at[orow], sem_out))
    for cp in out_cps:
        cp.wait()


@jax.jit
def kernel(instance_ids, eps, weight_mu, weight_logvar):
    out = jax.ShapeDtypeStruct((B, D), jnp.float32)
    f = pl.kernel(
        _sc_body,
        mesh=plsc.VectorSubcoreMesh(core_axis_name="c", subcore_axis_name="s"),
        compiler_params=pltpu.CompilerParams(use_tc_tiling_on_sc=False),
        out_type=[out, out, out],
        scratch_types=[
            pltpu.VMEM((NCHUNK, CHUNK), jnp.int32),
            pltpu.VMEM((BPW, D), jnp.float32),
            pltpu.VMEM((BPW, D), jnp.float32),
            pltpu.VMEM((BPW, D), jnp.float32),
            pltpu.VMEM((BPW, D), jnp.float32),
            pltpu.SemaphoreType.DMA,
            pltpu.SemaphoreType.DMA,
            pltpu.SemaphoreType.DMA,
            pltpu.SemaphoreType.DMA,
        ],
    )
    lat, mu, lv = f(instance_ids, eps, weight_mu, weight_logvar)
    return (lat, mu, lv)


# speedup vs baseline: 3.8289x; 3.8289x over previous
import functools

import jax
import jax.numpy as jnp
from jax import lax
from jax.experimental import pallas as pl
from jax.experimental.pallas import tpu as pltpu
from jax.experimental.pallas import tpu_sc as plsc

N = 1000000
B = 16384
D = 32
L = 16
TS = 8
TL = 128
SLABS = D // TS
NBUF = 4

_info = plsc.get_sparse_core_info()
NC = _info.num_cores
NS = _info.num_subcores
NW = NC * NS
BPW = B // NW
NVEC = BPW // L
WCH = BPW // TL


def _sc_body(ids_hbm, epsT_hbm, muT_hbm, lvT_hbm,
             latT_out, muT_o, lvT_o,
             idx_v, eps_v, lat_v, muo_v, lvo_v,
             mu_t0, mu_t1, mu_t2, mu_t3,
             lv_t0, lv_t1, lv_t2, lv_t3,
             sem_eps, sem_out,
             sem_mu0, sem_mu1, sem_mu2, sem_mu3,
             sem_lv0, sem_lv1, sem_lv2, sem_lv3):
    mu_t = [mu_t0, mu_t1, mu_t2, mu_t3]
    lv_t = [lv_t0, lv_t1, lv_t2, lv_t3]
    sem_mu = [sem_mu0, sem_mu1, sem_mu2, sem_mu3]
    sem_lv = [sem_lv0, sem_lv1, sem_lv2, sem_lv3]

    wid = lax.axis_index("s") * NC + lax.axis_index("c")
    base = pl.multiple_of(wid * BPW, BPW)

    pltpu.sync_copy(ids_hbm.at[pl.ds(base, BPW)], idx_v)
    eps_cps = [
        pltpu.async_copy(
            epsT_hbm.at[:, :, pl.ds(base + w * TL, TL)],
            eps_v.at[pl.ds(w * SLABS, SLABS)], sem_eps)
        for w in range(WCH)
    ]
    for cp in eps_cps:
        cp.wait()

    lane = lax.iota(jnp.int32, 16)

    def fire(k_c, b):
        cb = pl.multiple_of((k_c >> 7) * TL, TL)
        pltpu.async_copy(muT_hbm.at[:, :, pl.ds(cb, TL)], mu_t[b], sem_mu[b])
        pltpu.async_copy(lvT_hbm.at[:, :, pl.ds(cb, TL)], lv_t[b], sem_lv[b])

    def drain(b):
        pltpu.make_async_copy(muT_hbm.at[:, :, pl.ds(0, TL)], mu_t[b],
                              sem_mu[b]).wait()
        pltpu.make_async_copy(lvT_hbm.at[:, :, pl.ds(0, TL)], lv_t[b],
                              sem_lv[b]).wait()

    def process(k, k_c, b):
        cw = jnp.full((16,), k_c & (TL - 1), dtype=jnp.int32)
        kw = jnp.full((16,), (k >> 7) * SLABS, dtype=jnp.int32)
        kc = jnp.full((16,), k & (TL - 1), dtype=jnp.int32)
        for h in range(D // L):
            d = lane + h * L
            tr = lax.shift_right_logical(d, 3)
            r = lax.bitwise_and(d, 7)
            ow = kw + tr
            mu16 = plsc.load_gather(mu_t[b], [tr, r, cw])
            lv16 = plsc.load_gather(lv_t[b], [tr, r, cw])
            e16 = plsc.load_gather(eps_v, [ow, r, kc])
            std = jnp.exp(lv16 * 0.5)
            lat16 = mu16 + e16 * std
            plsc.store_scatter(lat_v, [ow, r, kc], lat16)
            plsc.store_scatter(muo_v, [ow, r, kc], mu16)
            plsc.store_scatter(lvo_v, [ow, r, kc], lv16)

    def body(v, carry):
        cm = list(carry)
        vec = idx_v[pl.ds(v * L, L)]
        for l in range(L):
            k = v * L + l
            c = vec[l]
            b = l % NBUF

            @pl.when(k >= NBUF)
            def _():
                drain(b)
                process(k - NBUF, cm[0], b)

            fire(c, b)
            cm = cm[1:] + [c]
        return tuple(cm)

    zero = jnp.int32(0)
    carry = lax.fori_loop(0, NVEC, body, (zero,) * NBUF)
    for i in range(NBUF):
        k = BPW - NBUF + i
        b = k % NBUF
        drain(b)
        process(k, carry[i], b)

    out_cps = []
    for w in range(WCH):
        src = pl.ds(w * SLABS, SLABS)
        dst = pl.ds(base + w * TL, TL)
        out_cps += [
            pltpu.async_copy(lat_v.at[src], latT_out.at[:, :, dst], sem_out),
            pltpu.async_copy(muo_v.at[src], muT_o.at[:, :, dst], sem_out),
            pltpu.async_copy(lvo_v.at[src], lvT_o.at[:, :, dst], sem_out),
        ]
    for cp in out_cps:
        cp.wait()


@jax.jit
def kernel(instance_ids, eps, weight_mu, weight_logvar):
    outT = jax.ShapeDtypeStruct((SLABS, TS, B), jnp.float32)
    tile_t = pltpu.VMEM((SLABS, TS, TL), jnp.float32)
    stage_t = pltpu.VMEM((WCH * SLABS, TS, TL), jnp.float32)
    f = pl.kernel(
        _sc_body,
        mesh=plsc.VectorSubcoreMesh(core_axis_name="c", subcore_axis_name="s"),
        compiler_params=pltpu.CompilerParams(use_tc_tiling_on_sc=True,
                                             needs_layout_passes=False),
        out_type=[outT, outT, outT],
        scratch_types=(
            [pltpu.VMEM((BPW,), jnp.int32), stage_t, stage_t, stage_t, stage_t]
            + [tile_t] * (2 * NBUF)
            + [pltpu.SemaphoreType.DMA] * (2 + 2 * NBUF)
        ),
    )
    epsT = eps.T.reshape(SLABS, TS, B)
    muT = weight_mu.T.reshape(SLABS, TS, N)
    lvT = weight_logvar.T.reshape(SLABS, TS, N)
    latT, muoT, lvoT = f(instance_ids, epsT, muT, lvT)
    unT = lambda x: x.reshape(D, B).T
    return (unT(latT), unT(muoT), unT(lvoT))


# speedup vs baseline: 3.9212x; 1.0241x over previous
import functools

import jax
import jax.numpy as jnp
from jax import lax
from jax.experimental import pallas as pl
from jax.experimental.pallas import tpu as pltpu
from jax.experimental.pallas import tpu_sc as plsc

N = 1000000
B = 16384
D = 32
L = 16
TS = 8
TL = 128
SLABS = D // TS
NBUF = 4

_info = plsc.get_sparse_core_info()
NC = _info.num_cores
NS = _info.num_subcores
NW = NC * NS
BPW = B // NW
NVEC = BPW // L
WCH = BPW // TL


def _sc_body(ids_hbm, epsT_hbm, muT_hbm, lvT_hbm,
             latT_out, muT_o, lvT_o,
             idx_v, eps_v, lat_v, muo_v, lvo_v,
             mu_t0, mu_t1, mu_t2, mu_t3,
             lv_t0, lv_t1, lv_t2, lv_t3,
             sem_eps, sem_out,
             sem_mu0, sem_mu1, sem_mu2, sem_mu3,
             sem_lv0, sem_lv1, sem_lv2, sem_lv3):
    mu_t = [mu_t0, mu_t1, mu_t2, mu_t3]
    lv_t = [lv_t0, lv_t1, lv_t2, lv_t3]
    sem_mu = [sem_mu0, sem_mu1, sem_mu2, sem_mu3]
    sem_lv = [sem_lv0, sem_lv1, sem_lv2, sem_lv3]

    wid = lax.axis_index("s") * NC + lax.axis_index("c")
    base = pl.multiple_of(wid * BPW, BPW)

    pltpu.sync_copy(ids_hbm.at[pl.ds(base, BPW)], idx_v)
    eps_cps = [
        pltpu.async_copy(
            epsT_hbm.at[:, :, pl.ds(base + w * TL, TL)],
            eps_v.at[pl.ds(w * SLABS, SLABS)], sem_eps)
        for w in range(WCH)
    ]
    for cp in eps_cps:
        cp.wait()

    lane = lax.iota(jnp.int32, 16)

    def fire(k_c, b):
        cb = pl.multiple_of((k_c >> 7) * TL, TL)
        pltpu.async_copy(muT_hbm.at[:, :, pl.ds(cb, TL)], mu_t[b], sem_mu[b])
        pltpu.async_copy(lvT_hbm.at[:, :, pl.ds(cb, TL)], lv_t[b], sem_lv[b])

    def drain(b):
        pltpu.make_async_copy(muT_hbm.at[:, :, pl.ds(0, TL)], mu_t[b],
                              sem_mu[b]).wait()
        pltpu.make_async_copy(lvT_hbm.at[:, :, pl.ds(0, TL)], lv_t[b],
                              sem_lv[b]).wait()

    def process(k, k_c, b):
        cw = jnp.full((16,), k_c & (TL - 1), dtype=jnp.int32)
        kw = jnp.full((16,), (k >> 7) * SLABS, dtype=jnp.int32)
        kc = jnp.full((16,), k & (TL - 1), dtype=jnp.int32)
        for h in range(D // L):
            d = lane + h * L
            tr = lax.shift_right_logical(d, 3)
            r = lax.bitwise_and(d, 7)
            ow = kw + tr
            mu16 = plsc.load_gather(mu_t[b], [tr, r, cw])
            lv16 = plsc.load_gather(lv_t[b], [tr, r, cw])
            e16 = plsc.load_gather(eps_v, [ow, r, kc])
            std = jnp.exp(lv16 * 0.5)
            lat16 = mu16 + e16 * std
            plsc.store_scatter(lat_v, [ow, r, kc], lat16)
            plsc.store_scatter(muo_v, [ow, r, kc], mu16)
            plsc.store_scatter(lvo_v, [ow, r, kc], lv16)

    def body(v, carry):
        cm = list(carry)
        vec = idx_v[pl.ds(v * L, L)]
        for l in range(L):
            k = v * L + l
            c = vec[l]
            b = l % NBUF

            @pl.when(k >= NBUF)
            def _():
                drain(b)
                km = k - NBUF
                cw = jnp.full((16,), cm[0] & (TL - 1), dtype=jnp.int32)
                mus, lvs = [], []
                for h in range(D // L):
                    d = lane + h * L
                    tr = lax.shift_right_logical(d, 3)
                    r = lax.bitwise_and(d, 7)
                    mus.append(plsc.load_gather(mu_t[b], [tr, r, cw]))
                    lvs.append(plsc.load_gather(lv_t[b], [tr, r, cw]))
                fire(c, b)
                kw = jnp.full((16,), (km >> 7) * SLABS, dtype=jnp.int32)
                kc = jnp.full((16,), km & (TL - 1), dtype=jnp.int32)
                for h in range(D // L):
                    d = lane + h * L
                    tr = lax.shift_right_logical(d, 3)
                    r = lax.bitwise_and(d, 7)
                    ow = kw + tr
                    e16 = plsc.load_gather(eps_v, [ow, r, kc])
                    lat16 = mus[h] + e16 * jnp.exp(lvs[h] * 0.5)
                    plsc.store_scatter(lat_v, [ow, r, kc], lat16)
                    plsc.store_scatter(muo_v, [ow, r, kc], mus[h])
                    plsc.store_scatter(lvo_v, [ow, r, kc], lvs[h])

            @pl.when(k < NBUF)
            def _():
                fire(c, b)

            cm = cm[1:] + [c]
        return tuple(cm)

    zero = jnp.int32(0)
    carry = lax.fori_loop(0, NVEC, body, (zero,) * NBUF)
    for i in range(NBUF):
        k = BPW - NBUF + i
        b = k % NBUF
        drain(b)
        process(k, carry[i], b)

    out_cps = []
    for w in range(WCH):
        src = pl.ds(w * SLABS, SLABS)
        dst = pl.ds(base + w * TL, TL)
        out_cps += [
            pltpu.async_copy(lat_v.at[src], latT_out.at[:, :, dst], sem_out),
            pltpu.async_copy(muo_v.at[src], muT_o.at[:, :, dst], sem_out),
            pltpu.async_copy(lvo_v.at[src], lvT_o.at[:, :, dst], sem_out),
        ]
    for cp in out_cps:
        cp.wait()


@jax.jit
def kernel(instance_ids, eps, weight_mu, weight_logvar):
    outT = jax.ShapeDtypeStruct((SLABS, TS, B), jnp.float32)
    tile_t = pltpu.VMEM((SLABS, TS, TL), jnp.float32)
    stage_t = pltpu.VMEM((WCH * SLABS, TS, TL), jnp.float32)
    f = pl.kernel(
        _sc_body,
        mesh=plsc.VectorSubcoreMesh(core_axis_name="c", subcore_axis_name="s"),
        compiler_params=pltpu.CompilerParams(use_tc_tiling_on_sc=True,
                                             needs_layout_passes=False),
        out_type=[outT, outT, outT],
        scratch_types=(
            [pltpu.VMEM((BPW,), jnp.int32), stage_t, stage_t, stage_t, stage_t]
            + [tile_t] * (2 * NBUF)
            + [pltpu.SemaphoreType.DMA] * (2 + 2 * NBUF)
        ),
    )
    epsT = eps.T.reshape(SLABS, TS, B)
    muT = weight_mu.T.reshape(SLABS, TS, N)
    lvT = weight_logvar.T.reshape(SLABS, TS, N)
    latT, muoT, lvoT = f(instance_ids, epsT, muT, lvT)
    unT = lambda x: x.reshape(D, B).T
    return (unT(latT), unT(muoT), unT(lvoT))
